# Initial kernel scaffold; baseline (speedup 1.0000x reference)
import jax
import jax.numpy as jnp
from jax.experimental import pallas as pl


def kernel(x, W0, W1, W2, W3, W4, W5, W6, W7, W8, W9):
    raise NotImplementedError("write your pallas kernel here")


# speedup vs baseline: 106.4014x; 106.4014x over previous
import functools

import jax
import jax.numpy as jnp
from jax import lax
from jax.experimental import pallas as pl
from jax.experimental.pallas import tpu as pltpu
from jax.experimental.pallas import tpu_sc as plsc

BS = 4096
N_VARS = 26
IN_LEN = 200
N_CAT = 10
ROW = N_VARS * IN_LEN
GROW = N_CAT * IN_LEN
NC, NS = 2, 16
NW = NC * NS
ROWS_PER_W = BS // NW
CHUNK = 8
N_CHUNKS = ROWS_PER_W // CHUNK


def _sc_embed(x_flat, table, chan_off):
    mesh = plsc.VectorSubcoreMesh(core_axis_name="c", subcore_axis_name="s")

    @functools.partial(
        pl.kernel,
        mesh=mesh,
        out_type=jax.ShapeDtypeStruct((BS * ROW,), jnp.float32),
        scratch_types=[
            pltpu.VMEM((N_CAT * 1000,), jnp.float32),
            pltpu.VMEM((GROW,), jnp.int32),
            pltpu.VMEM((CHUNK * ROW,), jnp.float32),
        ],
        compiler_params=pltpu.CompilerParams(needs_layout_passes=False),
    )
    def k(x_hbm, tab_hbm, off_hbm, out_hbm, tab_v, off_v, buf_v):
        wid = lax.axis_index("s") * NC + lax.axis_index("c")
        pltpu.sync_copy(tab_hbm, tab_v)
        pltpu.sync_copy(off_hbm, off_v)
        base = wid * (ROWS_PER_W * ROW)

        def chunk_body(ci, carry):
            off = base + ci * (CHUNK * ROW)
            pltpu.sync_copy(x_hbm.at[pl.ds(off, CHUNK * ROW)], buf_v)

            def row_body(r, c2):
                rbase = r * ROW
                for i in range(GROW // 16):
                    sl = pl.ds(rbase + i * 16, 16)
                    ids = buf_v[sl].astype(jnp.int32)
                    idx = ids + off_v[pl.ds(i * 16, 16)]
                    buf_v[sl] = plsc.load_gather(tab_v, [idx])
                return c2

            lax.fori_loop(0, CHUNK, row_body, 0)
            pltpu.sync_copy(buf_v, out_hbm.at[pl.ds(off, CHUNK * ROW)])
            return carry

        lax.fori_loop(0, N_CHUNKS, chunk_body, 0)

    return k(x_flat, table, chan_off)


def kernel(x, W0, W1, W2, W3, W4, W5, W6, W7, W8, W9):
    table = jnp.concatenate(
        [W0, W1, W2, W3, W4, W5, W6, W7, W8, W9], axis=0
    )[:, 0]
    chan_off = (jnp.arange(GROW, dtype=jnp.int32) // IN_LEN) * 1000
    out = _sc_embed(x.reshape(BS * ROW), table, chan_off)
    return out.reshape(BS, N_VARS, IN_LEN)


# speedup vs baseline: 132.8601x; 1.2487x over previous
import functools

import jax
import jax.numpy as jnp
from jax import lax
from jax.experimental import pallas as pl
from jax.experimental.pallas import tpu as pltpu
from jax.experimental.pallas import tpu_sc as plsc

BS = 4096
N_VARS = 26
IN_LEN = 200
N_CAT = 10
ROW = N_VARS * IN_LEN
NC, NS = 2, 16
NW = NC * NS
ROWS_PER_W = BS // NW
CHUNK = 4
N_CHUNKS = ROWS_PER_W // CHUNK
CWORDS = CHUNK * ROW
NBUF = 3
TSLICES = list(range(0, IN_LEN - 32, 16))
TLAST = (IN_LEN - 24, IN_LEN - 16)


def _sc_embed(x_flat, table):
    mesh = plsc.VectorSubcoreMesh(core_axis_name="c", subcore_axis_name="s")

    @functools.partial(
        pl.kernel,
        mesh=mesh,
        out_type=jax.ShapeDtypeStruct((BS * ROW,), jnp.float32),
        scratch_types=[
            pltpu.VMEM((N_CAT * 1000,), jnp.float32),
            pltpu.VMEM((CWORDS,), jnp.float32),
            pltpu.VMEM((CWORDS,), jnp.float32),
            pltpu.VMEM((CWORDS,), jnp.float32),
            pltpu.SemaphoreType.DMA,
            pltpu.SemaphoreType.DMA,
            pltpu.SemaphoreType.DMA,
            pltpu.SemaphoreType.DMA,
            pltpu.SemaphoreType.DMA,
            pltpu.SemaphoreType.DMA,
        ],
        compiler_params=pltpu.CompilerParams(needs_layout_passes=False),
    )
    def k(x_hbm, tab_hbm, out_hbm, tab_v, buf0, buf1, buf2,
          si0, si1, si2, so0, so1, so2):
        wid = lax.axis_index("s") * NC + lax.axis_index("c")
        pltpu.sync_copy(tab_hbm, tab_v)
        base = wid * (ROWS_PER_W * ROW)
        bufs = (buf0, buf1, buf2)
        sis, sos = (si0, si1, si2), (so0, so1, so2)

        def issue_in(ci, b):
            pltpu.async_copy(
                x_hbm.at[pl.ds(base + ci * CWORDS, CWORDS)], bufs[b], sis[b]
            )

        def issue_out(ci, b):
            pltpu.async_copy(
                bufs[b], out_hbm.at[pl.ds(base + ci * CWORDS, CWORDS)], sos[b]
            )

        def wait_in(b):
            pltpu.make_async_copy(
                x_hbm.at[pl.ds(0, CWORDS)], bufs[b], sis[b]
            ).wait()

        def wait_out(b):
            pltpu.make_async_copy(
                bufs[b], out_hbm.at[pl.ds(0, CWORDS)], sos[b]
            ).wait()

        def compute(b):
            buf = bufs[b]

            def row_body(r, c2):
                rb = r * ROW

                def chan_body(c, c3):
                    cb = rb + c * IN_LEN
                    coff = c * 1000
                    for t in TSLICES:
                        sl = pl.ds(cb + t, 16)
                        idx = buf[sl].astype(jnp.int32) + coff
                        buf[sl] = plsc.load_gather(tab_v, [idx])
                    sla = pl.ds(cb + TLAST[0], 16)
                    slb = pl.ds(cb + TLAST[1], 16)
                    ia = buf[sla].astype(jnp.int32) + coff
                    ib = buf[slb].astype(jnp.int32) + coff
                    ga = plsc.load_gather(tab_v, [ia])
                    gb = plsc.load_gather(tab_v, [ib])
                    buf[sla] = ga
                    buf[slb] = gb
                    return c3

                lax.fori_loop(0, N_CAT, chan_body, 0)
                return c2

            lax.fori_loop(0, CHUNK, row_body, 0)

        issue_in(0, 0)
        issue_in(1, 1)
        for ci in range(N_CHUNKS):
            b = ci % NBUF
            wait_in(b)
            compute(b)
            issue_out(ci, b)
            if ci + 2 < N_CHUNKS:
                nb = (ci + 2) % NBUF
                if ci >= 1:
                    wait_out(nb)
                issue_in(ci + 2, nb)
        for ci in range(N_CHUNKS - NBUF, N_CHUNKS):
            wait_out(ci % NBUF)

    return k(x_flat, table)


def kernel(x, W0, W1, W2, W3, W4, W5, W6, W7, W8, W9):
    table = jnp.concatenate(
        [W0, W1, W2, W3, W4, W5, W6, W7, W8, W9], axis=0
    )[:, 0]
    out = _sc_embed(x.reshape(BS * ROW), table)
    return out.reshape(BS, N_VARS, IN_LEN)


# speedup vs baseline: 136.8623x; 1.0301x over previous
import functools

import jax
import jax.numpy as jnp
from jax import lax
from jax.experimental import pallas as pl
from jax.experimental.pallas import tpu as pltpu
from jax.experimental.pallas import tpu_sc as plsc

BS = 4096
N_VARS = 26
IN_LEN = 200
N_CAT = 10
ROW = N_VARS * IN_LEN
NC, NS = 2, 16
NW = NC * NS
ROWS_PER_W = BS // NW
CHUNK = 4
N_CHUNKS = ROWS_PER_W // CHUNK
CWORDS = CHUNK * ROW
NBUF = 3
TSLICES = list(range(0, IN_LEN - 32, 16))
TLAST = (IN_LEN - 24, IN_LEN - 16)
FMAGIC = jnp.float32(2.0 ** 23)
FBIAS = jnp.int32(0x4B000000)


def _sc_embed(x_flat, table):
    mesh = plsc.VectorSubcoreMesh(core_axis_name="c", subcore_axis_name="s")

    @functools.partial(
        pl.kernel,
        mesh=mesh,
        out_type=jax.ShapeDtypeStruct((BS * ROW,), jnp.float32),
        scratch_types=[
            pltpu.VMEM((N_CAT * 1000,), jnp.float32),
            pltpu.VMEM((CWORDS,), jnp.float32),
            pltpu.VMEM((CWORDS,), jnp.float32),
            pltpu.VMEM((CWORDS,), jnp.float32),
            pltpu.SemaphoreType.DMA,
            pltpu.SemaphoreType.DMA,
            pltpu.SemaphoreType.DMA,
            pltpu.SemaphoreType.DMA,
            pltpu.SemaphoreType.DMA,
            pltpu.SemaphoreType.DMA,
        ],
        compiler_params=pltpu.CompilerParams(needs_layout_passes=False),
    )
    def k(x_hbm, tab_hbm, out_hbm, tab_v, buf0, buf1, buf2,
          si0, si1, si2, so0, so1, so2):
        wid = lax.axis_index("s") * NC + lax.axis_index("c")
        pltpu.sync_copy(tab_hbm, tab_v)
        base = wid * (ROWS_PER_W * ROW)
        bufs = (buf0, buf1, buf2)
        sis, sos = (si0, si1, si2), (so0, so1, so2)

        def issue_in(ci, b):
            pltpu.async_copy(
                x_hbm.at[pl.ds(base + ci * CWORDS, CWORDS)], bufs[b], sis[b]
            )

        def issue_out(ci, b):
            pltpu.async_copy(
                bufs[b], out_hbm.at[pl.ds(base + ci * CWORDS, CWORDS)], sos[b]
            )

        def wait_in(b):
            pltpu.make_async_copy(
                x_hbm.at[pl.ds(0, CWORDS)], bufs[b], sis[b]
            ).wait()

        def wait_out(b):
            pltpu.make_async_copy(
                bufs[b], out_hbm.at[pl.ds(0, CWORDS)], sos[b]
            ).wait()

        def compute(b):
            buf = bufs[b]

            def rc_body(rc, c2):
                r = rc // N_CAT
                c = rc - r * N_CAT
                cb = r * ROW + c * IN_LEN
                bias = FBIAS - c * 1000

                def do(sl):
                    v = buf[sl] + FMAGIC
                    return plsc.load_gather(
                        tab_v, [plsc.bitcast(v, jnp.int32) - bias]
                    )

                for t in TSLICES:
                    sl = pl.ds(cb + t, 16)
                    buf[sl] = do(sl)
                sla = pl.ds(cb + TLAST[0], 16)
                slb = pl.ds(cb + TLAST[1], 16)
                ga = do(sla)
                gb = do(slb)
                buf[sla] = ga
                buf[slb] = gb
                return c2

            lax.fori_loop(0, CHUNK * N_CAT, rc_body, 0)

        issue_in(0, 0)
        issue_in(1, 1)
        for ci in range(N_CHUNKS):
            b = ci % NBUF
            wait_in(b)
            compute(b)
            issue_out(ci, b)
            if ci + 2 < N_CHUNKS:
                nb = (ci + 2) % NBUF
                if ci >= 1:
                    wait_out(nb)
                issue_in(ci + 2, nb)
        for ci in range(N_CHUNKS - NBUF, N_CHUNKS):
            wait_out(ci % NBUF)

    return k(x_flat, table)


def kernel(x, W0, W1, W2, W3, W4, W5, W6, W7, W8, W9):
    table = jnp.concatenate(
        [W0, W1, W2, W3, W4, W5, W6, W7, W8, W9], axis=0
    )[:, 0]
    out = _sc_embed(x.reshape(BS * ROW), table)
    return out.reshape(BS, N_VARS, IN_LEN)
